# Initial kernel scaffold; baseline (speedup 1.0000x reference)
import jax
import jax.numpy as jnp
from jax.experimental import pallas as pl


def kernel(n_feat, e_feat, edge_index, W_node0, b_node0, W_ni0, W_nj0, W_fij0, attn0, b_edge0, W_node1, b_node1, W_ni1, W_nj1, W_fij1, attn1, b_edge1):
    raise NotImplementedError("write your pallas kernel here")


# speedup vs baseline: 22.7143x; 22.7143x over previous
import functools

import jax
import jax.numpy as jnp
import numpy as np
from jax import lax
from jax.experimental import pallas as pl
from jax.experimental.pallas import tpu as pltpu
from jax.experimental.pallas import tpu_sc as plsc

N = 10000
E = 320000
D = 128
DE = 16
H = 8
FE = 8
FN = 16
HFE = H * FE
HFN = H * FN

NC = 2
NS = 16
NW = NC * NS
CHUNK = 128
ROUNDS = 79
EW = CHUNK * ROUNDS
EPAD = EW * NW
ROWS_PER_SUB = N // NS

_MESH = plsc.VectorSubcoreMesh(
    core_axis_name="c", subcore_axis_name="s", num_cores=NC, num_subcores=NS)

f32 = jnp.float32
i32 = jnp.int32


def _mm_body(k2, hin_ref, wni_ref, wnj_ref, wn_ref, bn_ref,
             fni_ref, fnj_ref, h_ref):
    x = hin_ref[0]
    for k in range(1, k2):
        x = x + hin_ref[k]
    fni_ref[...] = jnp.dot(x, wni_ref[...], preferred_element_type=f32)
    fnj_ref[...] = jnp.dot(x, wnj_ref[...], preferred_element_type=f32)
    h_ref[...] = jnp.dot(x, wn_ref[...], preferred_element_type=f32) + bn_ref[...]


def _node_matmuls(hin, Wni, Wnj, Wn, bn):
    k2 = hin.shape[0]
    bnrows = 400
    grid = (N // bnrows,)
    return pl.pallas_call(
        functools.partial(_mm_body, k2),
        grid=grid,
        in_specs=[
            pl.BlockSpec((k2, bnrows, D), lambda i: (0, i, 0)),
            pl.BlockSpec((D, HFE), lambda i: (0, 0)),
            pl.BlockSpec((D, HFE), lambda i: (0, 0)),
            pl.BlockSpec((D, HFN), lambda i: (0, 0)),
            pl.BlockSpec((1, HFN), lambda i: (0, 0)),
        ],
        out_specs=[
            pl.BlockSpec((bnrows, HFE), lambda i: (i, 0)),
            pl.BlockSpec((bnrows, HFE), lambda i: (i, 0)),
            pl.BlockSpec((bnrows, HFN), lambda i: (i, 0)),
        ],
        out_shape=[
            jax.ShapeDtypeStruct((N, HFE), f32),
            jax.ShapeDtypeStruct((N, HFE), f32),
            jax.ShapeDtypeStruct((N, HFN), f32),
        ],
    )(hin, Wni, Wnj, Wn, bn)


def _edge_body(be, ga_ref, gb_ref, ef_ref, wf_ref, bias_ref, attn_ref, g_ref,
               out_ref):
    z = ga_ref[...] + gb_ref[...]
    z = z + jnp.dot(ef_ref[...], wf_ref[...], preferred_element_type=f32)
    z = z + bias_ref[...]
    z = jnp.where(z >= 0, z, 0.01 * z)
    e = jnp.dot(z * attn_ref[...], g_ref[...], preferred_element_type=f32)
    pid = pl.program_id(0)
    eid = pid * be + lax.broadcasted_iota(i32, (be, H), 0)
    out_ref[...] = jnp.where(eid < E, jnp.exp(e), 0.0)


def _edge_eexp(ga, gb, efp, Wfij, b_edge, attn_flat, G):
    be = 1024
    grid = (EPAD // be,)
    return pl.pallas_call(
        functools.partial(_edge_body, be),
        grid=grid,
        in_specs=[
            pl.BlockSpec((be, HFE), lambda i: (i, 0)),
            pl.BlockSpec((be, HFE), lambda i: (i, 0)),
            pl.BlockSpec((be, DE), lambda i: (i, 0)),
            pl.BlockSpec((DE, HFE), lambda i: (0, 0)),
            pl.BlockSpec((1, HFE), lambda i: (0, 0)),
            pl.BlockSpec((1, HFE), lambda i: (0, 0)),
            pl.BlockSpec((HFE, H), lambda i: (0, 0)),
        ],
        out_specs=pl.BlockSpec((be, H), lambda i: (i, 0)),
        out_shape=jax.ShapeDtypeStruct((EPAD, H), f32),
    )(ga, gb, efp, Wfij, b_edge, attn_flat, G)


def _comb_body(p_ref, out_ref):
    s = p_ref[0] + p_ref[1]
    inv = 1.0 / (s + 1e-16)
    out_ref[...] = jnp.concatenate([inv, inv], axis=1)


def _combine_denom(parts):
    return pl.pallas_call(
        _comb_body,
        in_specs=[pl.BlockSpec((2, N, H), lambda: (0, 0, 0))],
        out_specs=pl.BlockSpec((N, 2 * H), lambda: (0, 0)),
        out_shape=jax.ShapeDtypeStruct((N, 2 * H), f32),
    )(parts)


def _fin_body(p_ref, out_ref):
    out_ref[...] = p_ref[0] + p_ref[1]


def _combine_parts(parts):
    bnrows = 2000
    return pl.pallas_call(
        _fin_body,
        grid=(N // bnrows,),
        in_specs=[pl.BlockSpec((2, bnrows, HFN), lambda i: (0, i, 0))],
        out_specs=pl.BlockSpec((bnrows, HFN), lambda i: (i, 0)),
        out_shape=jax.ShapeDtypeStruct((N, HFN), f32),
    )(parts)


def _gather_body(w, table, idx_hbm, out_hbm, idx_v, rows_v, sem):
    wid = lax.axis_index("s") * NC + lax.axis_index("c")

    def chunk(r, carry):
        eb = wid * EW + r * CHUNK
        pltpu.sync_copy(idx_hbm.at[pl.ds(eb, CHUNK)], idx_v)
        pltpu.async_copy(table.at[idx_v], rows_v, sem).wait()
        pltpu.sync_copy(rows_v, out_hbm.at[pl.ds(eb, CHUNK)])
        return carry

    lax.fori_loop(0, ROUNDS, chunk, 0)


def _gather_rows(table, idx, w):
    fn = pl.kernel(
        functools.partial(_gather_body, w),
        out_type=jax.ShapeDtypeStruct((EPAD, w), f32),
        mesh=_MESH,
        compiler_params=pltpu.CompilerParams(use_tc_tiling_on_sc=False),
        scratch_types=[
            pltpu.VMEM((CHUNK,), i32),
            pltpu.VMEM((CHUNK, w), f32),
            pltpu.SemaphoreType.DMA,
        ],
    )
    return fn(table, idx)


def _scat8_body(eexp_hbm, dst_hbm, zeros_hbm, out_hbm, di, ebuf, acc):
    c = lax.axis_index("c")
    s = lax.axis_index("s")
    wid = s * NC + c
    rb = s * ROWS_PER_SUB
    pltpu.sync_copy(zeros_hbm.at[pl.ds(rb, ROWS_PER_SUB)],
                    acc.at[pl.ds(rb, ROWS_PER_SUB)])
    plsc.subcore_barrier()

    def chunk(r, carry):
        eb = wid * EW + r * CHUNK
        pltpu.sync_copy(dst_hbm.at[pl.ds(eb, CHUNK)], di)
        pltpu.sync_copy(eexp_hbm.at[pl.ds(eb, CHUNK)], ebuf)
        pltpu.sync_copy(ebuf, acc.at[di], add=True)
        return carry

    lax.fori_loop(0, ROUNDS, chunk, 0)
    plsc.subcore_barrier()
    pltpu.sync_copy(acc.at[pl.ds(rb, ROWS_PER_SUB)],
                    out_hbm.at[c, pl.ds(rb, ROWS_PER_SUB)])


def _denom_partials(eexp, dstp, zeros8):
    fn = pl.kernel(
        _scat8_body,
        out_type=jax.ShapeDtypeStruct((NC, N, H), f32),
        mesh=_MESH,
        compiler_params=pltpu.CompilerParams(use_tc_tiling_on_sc=False),
        scratch_types=[
            pltpu.VMEM((CHUNK,), i32),
            pltpu.VMEM((CHUNK, H), f32),
            pltpu.VMEM_SHARED((N, H), f32),
        ],
    )
    return fn(eexp, dstp, zeros8)


def _msg_body(h_hbm, ivd_hbm, eexp2_hbm, src_hbm, dst_hbm, zeros_hbm, out_hbm,
              si, di, hb, ivb, eb_, msg, acc, sem1, sem2):
    c = lax.axis_index("c")
    s = lax.axis_index("s")
    wid = s * NC + c
    rb = s * ROWS_PER_SUB
    pltpu.sync_copy(zeros_hbm.at[pl.ds(rb, ROWS_PER_SUB)],
                    acc.at[pl.ds(rb, ROWS_PER_SUB)])
    plsc.subcore_barrier()

    lane = lax.broadcasted_iota(i32, (16,), 0)
    lo_half = lane < 8
    splats = [jnp.full((16, 1), k, i32) for k in range(16)]
    dnums = lax.GatherDimensionNumbers(
        offset_dims=(), collapsed_slice_dims=(0,), start_index_map=(0,))

    def _splat(v, idx):
        return lax.gather(v, idx, dnums, slice_sizes=(1,),
                          mode=lax.GatherScatterMode.PROMISE_IN_BOUNDS)

    def chunk(r, carry):
        eb = wid * EW + r * CHUNK
        pb = wid * (EW // 2) + r * (CHUNK // 2)
        pltpu.sync_copy(src_hbm.at[pl.ds(eb, CHUNK)], si)
        pltpu.sync_copy(dst_hbm.at[pl.ds(eb, CHUNK)], di)
        pltpu.sync_copy(eexp2_hbm.at[pl.ds(pb, CHUNK // 2)], eb_)
        cp1 = pltpu.async_copy(h_hbm.at[si], hb, sem1)
        cp2 = pltpu.async_copy(ivd_hbm.at[di], ivb, sem2)
        cp1.wait()
        cp2.wait()

        def pair(p, cc):
            ep = eb_[p, :]
            iv0 = ivb[2 * p, :]
            iv1 = ivb[2 * p + 1, :]
            ap = jnp.where(lo_half, iv0, iv1) * ep
            for k in range(H):
                b0 = _splat(ap, splats[k])
                msg[2 * p, pl.ds(k * 16, 16)] = hb[2 * p, pl.ds(k * 16, 16)] * b0
                b1 = _splat(ap, splats[8 + k])
                msg[2 * p + 1, pl.ds(k * 16, 16)] = (
                    hb[2 * p + 1, pl.ds(k * 16, 16)] * b1)
            return cc

        lax.fori_loop(0, CHUNK // 2, pair, 0)
        pltpu.sync_copy(msg, acc.at[di], add=True)
        return carry

    lax.fori_loop(0, ROUNDS, chunk, 0)
    plsc.subcore_barrier()
    pltpu.sync_copy(acc.at[pl.ds(rb, ROWS_PER_SUB)],
                    out_hbm.at[c, pl.ds(rb, ROWS_PER_SUB)])


def _message_partials(h, ivd, eexp2, srcp, dstp, zerosN):
    fn = pl.kernel(
        _msg_body,
        out_type=jax.ShapeDtypeStruct((NC, N, HFN), f32),
        mesh=_MESH,
        compiler_params=pltpu.CompilerParams(use_tc_tiling_on_sc=False),
        scratch_types=[
            pltpu.VMEM((CHUNK,), i32),
            pltpu.VMEM((CHUNK,), i32),
            pltpu.VMEM((CHUNK, HFN), f32),
            pltpu.VMEM((CHUNK, 2 * H), f32),
            pltpu.VMEM((CHUNK // 2, 16), f32),
            pltpu.VMEM((CHUNK, HFN), f32),
            pltpu.VMEM_SHARED((N, HFN), f32),
            pltpu.SemaphoreType.DMA,
            pltpu.SemaphoreType.DMA,
        ],
    )
    return fn(h, ivd, eexp2, srcp, dstp, zerosN)


def _layer(hin, efp, srcp, dstp, zeros8, zerosN, G,
           Wn, bn, Wni, Wnj, Wfij, attn, be_bias):
    fni, fnj, h = _node_matmuls(hin, Wni, Wnj, Wn, bn.reshape(1, HFN))
    ga = _gather_rows(fni, srcp, HFE)
    gb = _gather_rows(fnj, dstp, HFE)
    eexp = _edge_eexp(ga, gb, efp, Wfij, be_bias.reshape(1, HFE),
                      attn.reshape(1, HFE), G)
    dparts = _denom_partials(eexp, dstp, zeros8)
    ivd = _combine_denom(dparts)
    eexp2 = eexp.reshape(EPAD // 2, 2 * H)
    return _message_partials(h, ivd, eexp2, srcp, dstp, zerosN)


def kernel(n_feat, e_feat, edge_index, W_node0, b_node0, W_ni0, W_nj0,
           W_fij0, attn0, b_edge0, W_node1, b_node1, W_ni1, W_nj1, W_fij1,
           attn1, b_edge1):
    src = edge_index[0].astype(i32)
    dst = edge_index[1].astype(i32)
    pad = EPAD - E
    srcp = jnp.concatenate([src, jnp.zeros((pad,), i32)])
    dstp = jnp.concatenate([dst, jnp.zeros((pad,), i32)])
    efp = jnp.concatenate([e_feat, jnp.zeros((pad, DE), f32)])
    zeros8 = jnp.zeros((N, H), f32)
    zerosN = jnp.zeros((N, HFN), f32)
    G = jnp.asarray(np.kron(np.eye(H), np.ones((FE, 1))), f32)

    parts0 = _layer(n_feat[None], efp, srcp, dstp, zeros8, zerosN, G,
                    W_node0, b_node0, W_ni0, W_nj0, W_fij0, attn0, b_edge0)
    parts1 = _layer(parts0, efp, srcp, dstp, zeros8, zerosN, G,
                    W_node1, b_node1, W_ni1, W_nj1, W_fij1, attn1, b_edge1)
    return _combine_parts(parts1)


# speedup vs baseline: 30.0772x; 1.3242x over previous
import functools

import jax
import jax.numpy as jnp
import numpy as np
from jax import lax
from jax.experimental import pallas as pl
from jax.experimental.pallas import tpu as pltpu
from jax.experimental.pallas import tpu_sc as plsc

N = 10000
E = 320000
D = 128
DE = 16
H = 8
FE = 8
FN = 16
HFE = H * FE
HFN = H * FN

NC = 2
NS = 16
NW = NC * NS
CHUNK = 128
UNITS = 80
EW = CHUNK * UNITS
EPAD = EW * NW
NROW = EPAD // CHUNK
ROWS_PER_SUB = N // NS
MUNITS = NROW // NS

_MESH = plsc.VectorSubcoreMesh(
    core_axis_name="c", subcore_axis_name="s", num_cores=NC, num_subcores=NS)
_SC_PARAMS = pltpu.CompilerParams(use_tc_tiling_on_sc=False)

f32 = jnp.float32
i32 = jnp.int32


def _wid():
    return lax.axis_index("s") * NC + lax.axis_index("c")


def _mm_body(colsplit, hin_ref, wni_ref, wnj_ref, wn_ref, bn_ref,
             fni_ref, fnj_ref, hlo_ref, hhi_ref):
    if colsplit:
        x = jnp.concatenate([hin_ref[0], hin_ref[1]], axis=1)
    else:
        x = hin_ref[0]
    fni_ref[...] = jnp.dot(x, wni_ref[...], preferred_element_type=f32)
    fnj_ref[...] = jnp.dot(x, wnj_ref[...], preferred_element_type=f32)
    hfull = jnp.dot(x, wn_ref[...], preferred_element_type=f32) + bn_ref[...]
    hlo_ref[...] = hfull[:, :HFE]
    hhi_ref[...] = hfull[:, HFE:]


def _node_matmuls(hin, Wni, Wnj, Wn, bn):
    colsplit = hin.shape[0] == 2
    din = hin.shape[2]
    bnrows = 400
    grid = (N // bnrows,)
    return pl.pallas_call(
        functools.partial(_mm_body, colsplit),
        grid=grid,
        in_specs=[
            pl.BlockSpec((hin.shape[0], bnrows, din), lambda i: (0, i, 0)),
            pl.BlockSpec((D, HFE), lambda i: (0, 0)),
            pl.BlockSpec((D, HFE), lambda i: (0, 0)),
            pl.BlockSpec((D, HFN), lambda i: (0, 0)),
            pl.BlockSpec((1, HFN), lambda i: (0, 0)),
        ],
        out_specs=[
            pl.BlockSpec((bnrows, HFE), lambda i: (i, 0)),
            pl.BlockSpec((bnrows, HFE), lambda i: (i, 0)),
            pl.BlockSpec((bnrows, HFE), lambda i: (i, 0)),
            pl.BlockSpec((bnrows, HFE), lambda i: (i, 0)),
        ],
        out_shape=[
            jax.ShapeDtypeStruct((N, HFE), f32),
            jax.ShapeDtypeStruct((N, HFE), f32),
            jax.ShapeDtypeStruct((N, HFE), f32),
            jax.ShapeDtypeStruct((N, HFE), f32),
        ],
    )(hin, Wni, Wnj, Wn, bn)


def _edge_body(be, gs_ref, ef_ref, wf_ref, bias_ref, attn_ref, g_ref,
               out_ref):
    z = gs_ref[...]
    z = z + jnp.dot(ef_ref[...], wf_ref[...], preferred_element_type=f32)
    z = z + bias_ref[...]
    z = jnp.where(z >= 0, z, 0.01 * z)
    e = jnp.dot(z * attn_ref[...], g_ref[...], preferred_element_type=f32)
    pid = pl.program_id(0)
    eid = pid * be + lax.broadcasted_iota(i32, (be, H), 0)
    out_ref[...] = jnp.where(eid < E, jnp.exp(e), 0.0)


def _edge_eexp(gsum, efp, Wfij, b_edge, attn_flat, G):
    be = 1024
    grid = (EPAD // be,)
    return pl.pallas_call(
        functools.partial(_edge_body, be),
        grid=grid,
        in_specs=[
            pl.BlockSpec((be, HFE), lambda i: (i, 0)),
            pl.BlockSpec((be, DE), lambda i: (i, 0)),
            pl.BlockSpec((DE, HFE), lambda i: (0, 0)),
            pl.BlockSpec((1, HFE), lambda i: (0, 0)),
            pl.BlockSpec((1, HFE), lambda i: (0, 0)),
            pl.BlockSpec((HFE, H), lambda i: (0, 0)),
        ],
        out_specs=pl.BlockSpec((be, H), lambda i: (i, 0)),
        out_shape=jax.ShapeDtypeStruct((EPAD, H), f32),
    )(gsum, efp, Wfij, b_edge, attn_flat, G)


def _comb_body(p_ref, out_ref):
    s = p_ref[0] + p_ref[1]
    inv = 1.0 / (s + 1e-16)
    out_ref[...] = jnp.concatenate([inv, inv], axis=1)


def _combine_denom(parts):
    return pl.pallas_call(
        _comb_body,
        in_specs=[pl.BlockSpec((2, N, H), lambda: (0, 0, 0))],
        out_specs=pl.BlockSpec((N, 2 * H), lambda: (0, 0)),
        out_shape=jax.ShapeDtypeStruct((N, 2 * H), f32),
    )(parts)


def _fin_body(p_ref, out_ref):
    out_ref[...] = jnp.concatenate([p_ref[0], p_ref[1]], axis=1)


def _combine_parts(parts):
    bnrows = 2000
    return pl.pallas_call(
        _fin_body,
        grid=(N // bnrows,),
        in_specs=[pl.BlockSpec((2, bnrows, HFE), lambda i: (0, i, 0))],
        out_specs=pl.BlockSpec((bnrows, HFN), lambda i: (i, 0)),
        out_shape=jax.ShapeDtypeStruct((N, HFN), f32),
    )(parts)


def _gsum_body(fni, fnj, src2d, dst2d, out_hbm,
               srcb, dstb, a0, a1, b0, b1, sg0, sg1):
    wid = _wid()
    rowb = wid * UNITS
    pltpu.sync_copy(src2d.at[pl.ds(rowb, UNITS)], srcb)
    pltpu.sync_copy(dst2d.at[pl.ds(rowb, UNITS)], dstb)
    abuf = (a0, a1)
    bbuf = (b0, b1)
    sg = (sg0, sg1)
    nr = UNITS // 2

    def descs(r, slot):
        u = 2 * r
        d = []
        for j in range(2):
            d.append(pltpu.make_async_copy(
                fni.at[srcb.at[u + j]],
                abuf[slot].at[pl.ds(j * CHUNK, CHUNK)], sg[slot]))
            d.append(pltpu.make_async_copy(
                fnj.at[dstb.at[u + j]],
                bbuf[slot].at[pl.ds(j * CHUNK, CHUNK)], sg[slot]))
        return d

    def issue(r, slot):
        for dsc in descs(r, slot):
            dsc.start()

    def process(r, slot):
        for dsc in descs(r, slot):
            dsc.wait()
        av = abuf[slot]
        bv = bbuf[slot]

        def addrow(i, c):
            for k in range(4):
                sl = pl.ds(k * 16, 16)
                av[i, sl] = av[i, sl] + bv[i, sl]
            return c

        lax.fori_loop(0, 2 * CHUNK, addrow, 0)
        eb = wid * EW + r * (2 * CHUNK)
        pltpu.sync_copy(av, out_hbm.at[pl.ds(eb, 2 * CHUNK)])

    issue(0, 0)
    issue(1, 1)

    def step(g, c):
        for s in range(2):
            r = 2 * g + s
            process(r, s)

            @pl.when(r + 2 < nr)
            def _():
                issue(r + 2, s)
        return c

    lax.fori_loop(0, nr // 2, step, 0)


def _gather_sum(fni, fnj, src2d, dst2d):
    fn = pl.kernel(
        _gsum_body,
        out_type=jax.ShapeDtypeStruct((EPAD, HFE), f32),
        mesh=_MESH,
        compiler_params=_SC_PARAMS,
        scratch_types=[
            pltpu.VMEM((UNITS, CHUNK), i32),
            pltpu.VMEM((UNITS, CHUNK), i32),
            pltpu.VMEM((2 * CHUNK, HFE), f32),
            pltpu.VMEM((2 * CHUNK, HFE), f32),
            pltpu.VMEM((2 * CHUNK, HFE), f32),
            pltpu.VMEM((2 * CHUNK, HFE), f32),
            pltpu.SemaphoreType.DMA,
            pltpu.SemaphoreType.DMA,
        ],
    )
    return fn(fni, fnj, src2d, dst2d)


def _scat8_body(eexp_hbm, dst2d, zeros_hbm, out_hbm, dstb, eb, acc):
    c = lax.axis_index("c")
    s = lax.axis_index("s")
    wid = s * NC + c
    rb = s * ROWS_PER_SUB
    pltpu.sync_copy(zeros_hbm.at[pl.ds(rb, ROWS_PER_SUB)],
                    acc.at[pl.ds(rb, ROWS_PER_SUB)])
    pltpu.sync_copy(dst2d.at[pl.ds(wid * UNITS, UNITS)], dstb)
    pltpu.sync_copy(eexp_hbm.at[pl.ds(wid * EW, EW)], eb)
    plsc.subcore_barrier()

    def unit(u, carry):
        pltpu.sync_copy(eb.at[pl.ds(u * CHUNK, CHUNK)],
                        acc.at[dstb.at[u]], add=True)
        return carry

    lax.fori_loop(0, UNITS, unit, 0)
    plsc.subcore_barrier()
    pltpu.sync_copy(acc.at[pl.ds(rb, ROWS_PER_SUB)],
                    out_hbm.at[c, pl.ds(rb, ROWS_PER_SUB)])


def _denom_partials(eexp, dst2d, zeros8):
    fn = pl.kernel(
        _scat8_body,
        out_type=jax.ShapeDtypeStruct((NC, N, H), f32),
        mesh=_MESH,
        compiler_params=_SC_PARAMS,
        scratch_types=[
            pltpu.VMEM((UNITS, CHUNK), i32),
            pltpu.VMEM((EW, H), f32),
            pltpu.VMEM_SHARED((N, H), f32),
        ],
    )
    return fn(eexp, dst2d, zeros8)


def _msg_body(hcat_hbm, ivd_hbm, eexp2_hbm, src2d, dst2d, zeros_hbm, out_hbm,
              srcb, dstb, hb0, hb1, ivb0, ivb1, ebf0, ebf1, mb0, mb1, acc,
              sh0, sh1, si0, si1, se0, se1):
    c = lax.axis_index("c")
    s = lax.axis_index("s")
    rb = s * ROWS_PER_SUB
    pltpu.sync_copy(zeros_hbm.at[pl.ds(rb, ROWS_PER_SUB)],
                    acc.at[pl.ds(rb, ROWS_PER_SUB)])
    rowb = s * MUNITS
    pltpu.sync_copy(src2d.at[pl.ds(rowb, MUNITS)], srcb)
    pltpu.sync_copy(dst2d.at[pl.ds(rowb, MUNITS)], dstb)
    coff = c * N

    def offrow(i, cc):
        for k in range(CHUNK // 16):
            sl = pl.ds(k * 16, 16)
            srcb[i, sl] = srcb[i, sl] + coff
        return cc

    lax.fori_loop(0, MUNITS, offrow, 0)
    plsc.subcore_barrier()

    hb = (hb0, hb1)
    ivb = (ivb0, ivb1)
    ebf = (ebf0, ebf1)
    mb = (mb0, mb1)
    sh = (sh0, sh1)
    si = (si0, si1)
    se = (se0, se1)

    lane = lax.broadcasted_iota(i32, (16,), 0)
    lo_half = lane < 8
    hbase = c * 4
    dnums = lax.GatherDimensionNumbers(
        offset_dims=(), collapsed_slice_dims=(0,), start_index_map=(0,))

    def _splat(v, k):
        idx = jnp.broadcast_to(jnp.reshape(k, (1, 1)), (16, 1))
        return lax.gather(v, idx, dnums, slice_sizes=(1,),
                          mode=lax.GatherScatterMode.PROMISE_IN_BOUNDS)

    def descs(u, slot):
        return [
            pltpu.make_async_copy(hcat_hbm.at[srcb.at[u]], hb[slot], sh[slot]),
            pltpu.make_async_copy(ivd_hbm.at[dstb.at[u]], ivb[slot], si[slot]),
            pltpu.make_async_copy(
                eexp2_hbm.at[pl.ds(rowb * (CHUNK // 2) + u * (CHUNK // 2),
                                   CHUNK // 2)],
                ebf[slot], se[slot]),
        ]

    def issue(u, slot):
        for dsc in descs(u, slot):
            dsc.start()

    def process(u, slot):
        for dsc in descs(u, slot):
            dsc.wait()
        hbv = hb[slot]
        ivv = ivb[slot]
        ebv = ebf[slot]
        mbv = mb[slot]

        def pair(p, cc):
            ep = ebv[p, :]
            iv0 = ivv[2 * p, :]
            iv1 = ivv[2 * p + 1, :]
            ap = jnp.where(lo_half, iv0, iv1) * ep
            for k in range(4):
                b0 = _splat(ap, hbase + k)
                sl = pl.ds(k * 16, 16)
                mbv[2 * p, sl] = hbv[2 * p, sl] * b0
                b1 = _splat(ap, hbase + (8 + k))
                mbv[2 * p + 1, sl] = hbv[2 * p + 1, sl] * b1
            return cc

        lax.fori_loop(0, CHUNK // 2, pair, 0)
        pltpu.sync_copy(mbv, acc.at[dstb.at[u]], add=True)

    issue(0, 0)
    issue(1, 1)

    def step(g, carry):
        for s2 in range(2):
            u = 2 * g + s2
            process(u, s2)

            @pl.when(u + 2 < MUNITS)
            def _():
                issue(u + 2, s2)
        return carry

    lax.fori_loop(0, MUNITS // 2, step, 0)
    plsc.subcore_barrier()
    pltpu.sync_copy(acc.at[pl.ds(rb, ROWS_PER_SUB)],
                    out_hbm.at[c, pl.ds(rb, ROWS_PER_SUB)])


def _message_partials(hcat, ivd, eexp2, src2d, dst2d, zerosN):
    fn = pl.kernel(
        _msg_body,
        out_type=jax.ShapeDtypeStruct((NC, N, HFE), f32),
        mesh=_MESH,
        compiler_params=_SC_PARAMS,
        scratch_types=[
            pltpu.VMEM((MUNITS, CHUNK), i32),
            pltpu.VMEM((MUNITS, CHUNK), i32),
            pltpu.VMEM((CHUNK, HFE), f32),
            pltpu.VMEM((CHUNK, HFE), f32),
            pltpu.VMEM((CHUNK, 2 * H), f32),
            pltpu.VMEM((CHUNK, 2 * H), f32),
            pltpu.VMEM((CHUNK // 2, 16), f32),
            pltpu.VMEM((CHUNK // 2, 16), f32),
            pltpu.VMEM((CHUNK, HFE), f32),
            pltpu.VMEM((CHUNK, HFE), f32),
            pltpu.VMEM_SHARED((N, HFE), f32),
            pltpu.SemaphoreType.DMA,
            pltpu.SemaphoreType.DMA,
            pltpu.SemaphoreType.DMA,
            pltpu.SemaphoreType.DMA,
            pltpu.SemaphoreType.DMA,
            pltpu.SemaphoreType.DMA,
        ],
    )
    return fn(hcat, ivd, eexp2, src2d, dst2d, zerosN)


def _layer(hin, efp, src2d, dst2d, zeros8, zerosN, G,
           Wn, bn, Wni, Wnj, Wfij, attn, be_bias):
    fni, fnj, h_lo, h_hi = _node_matmuls(hin, Wni, Wnj, Wn, bn.reshape(1, HFN))
    hcat = jnp.concatenate([h_lo, h_hi], axis=0)
    gsum = _gather_sum(fni, fnj, src2d, dst2d)
    eexp = _edge_eexp(gsum, efp, Wfij, be_bias.reshape(1, HFE),
                      attn.reshape(1, HFE), G)
    dparts = _denom_partials(eexp, dst2d, zeros8)
    ivd = _combine_denom(dparts)
    eexp2 = eexp.reshape(EPAD // 2, 2 * H)
    return _message_partials(hcat, ivd, eexp2, src2d, dst2d, zerosN)


def kernel(n_feat, e_feat, edge_index, W_node0, b_node0, W_ni0, W_nj0,
           W_fij0, attn0, b_edge0, W_node1, b_node1, W_ni1, W_nj1, W_fij1,
           attn1, b_edge1):
    src = edge_index[0].astype(i32)
    dst = edge_index[1].astype(i32)
    pad = EPAD - E
    src2d = jnp.concatenate([src, jnp.zeros((pad,), i32)]).reshape(NROW, CHUNK)
    dst2d = jnp.concatenate([dst, jnp.zeros((pad,), i32)]).reshape(NROW, CHUNK)
    efp = jnp.concatenate([e_feat, jnp.zeros((pad, DE), f32)])
    zeros8 = jnp.zeros((N, H), f32)
    zerosN = jnp.zeros((N, HFE), f32)
    G = jnp.asarray(np.kron(np.eye(H), np.ones((FE, 1))), f32)

    parts0 = _layer(n_feat[None], efp, src2d, dst2d, zeros8, zerosN, G,
                    W_node0, b_node0, W_ni0, W_nj0, W_fij0, attn0, b_edge0)
    parts1 = _layer(parts0, efp, src2d, dst2d, zeros8, zerosN, G,
                    W_node1, b_node1, W_ni1, W_nj1, W_fij1, attn1, b_edge1)
    return _combine_parts(parts1)


# speedup vs baseline: 30.4410x; 1.0121x over previous
import functools

import jax
import jax.numpy as jnp
import numpy as np
from jax import lax
from jax.experimental import pallas as pl
from jax.experimental.pallas import tpu as pltpu
from jax.experimental.pallas import tpu_sc as plsc

N = 10000
E = 320000
D = 128
DE = 16
H = 8
FE = 8
FN = 16
HFE = H * FE
HFN = H * FN

NC = 2
NS = 16
NW = NC * NS
CHUNK = 128
UNITS = 80
EW = CHUNK * UNITS
EPAD = EW * NW
NROW = EPAD // CHUNK
ROWS_PER_SUB = N // NS
MUNITS = NROW // NS

_MESH = plsc.VectorSubcoreMesh(
    core_axis_name="c", subcore_axis_name="s", num_cores=NC, num_subcores=NS)
_SC_PARAMS = pltpu.CompilerParams(use_tc_tiling_on_sc=False)

f32 = jnp.float32
i32 = jnp.int32


def _wid():
    return lax.axis_index("s") * NC + lax.axis_index("c")


def _mm_body(colsplit, hin_ref, wni_ref, wnj_ref, wn_ref, bn_ref,
             fni_ref, fnj_ref, hlo_ref, hhi_ref):
    if colsplit:
        x = jnp.concatenate([hin_ref[0], hin_ref[1]], axis=1)
    else:
        x = hin_ref[0]
    fni_ref[...] = jnp.dot(x, wni_ref[...], preferred_element_type=f32)
    fnj_ref[...] = jnp.dot(x, wnj_ref[...], preferred_element_type=f32)
    hfull = jnp.dot(x, wn_ref[...], preferred_element_type=f32) + bn_ref[...]
    hlo_ref[...] = hfull[:, :HFE]
    hhi_ref[...] = hfull[:, HFE:]


def _node_matmuls(hin, Wni, Wnj, Wn, bn):
    colsplit = hin.shape[0] == 2
    din = hin.shape[2]
    bnrows = 400
    grid = (N // bnrows,)
    return pl.pallas_call(
        functools.partial(_mm_body, colsplit),
        grid=grid,
        in_specs=[
            pl.BlockSpec((hin.shape[0], bnrows, din), lambda i: (0, i, 0)),
            pl.BlockSpec((D, HFE), lambda i: (0, 0)),
            pl.BlockSpec((D, HFE), lambda i: (0, 0)),
            pl.BlockSpec((D, HFN), lambda i: (0, 0)),
            pl.BlockSpec((1, HFN), lambda i: (0, 0)),
        ],
        out_specs=[
            pl.BlockSpec((bnrows, HFE), lambda i: (i, 0)),
            pl.BlockSpec((bnrows, HFE), lambda i: (i, 0)),
            pl.BlockSpec((bnrows, HFE), lambda i: (i, 0)),
            pl.BlockSpec((bnrows, HFE), lambda i: (i, 0)),
        ],
        out_shape=[
            jax.ShapeDtypeStruct((N, HFE), f32),
            jax.ShapeDtypeStruct((N, HFE), f32),
            jax.ShapeDtypeStruct((N, HFE), f32),
            jax.ShapeDtypeStruct((N, HFE), f32),
        ],
    )(hin, Wni, Wnj, Wn, bn)


def _edge_body(be, gs_ref, ef_ref, wf_ref, bias_ref, attn_ref, g_ref,
               out_ref):
    z = gs_ref[...]
    z = z + jnp.dot(ef_ref[...], wf_ref[...], preferred_element_type=f32)
    z = z + bias_ref[...]
    z = jnp.where(z >= 0, z, 0.01 * z)
    e = jnp.dot(z * attn_ref[...], g_ref[...], preferred_element_type=f32)
    pid = pl.program_id(0)
    eid = pid * be + lax.broadcasted_iota(i32, (be, H), 0)
    out_ref[...] = jnp.where(eid < E, jnp.exp(e), 0.0)


def _edge_eexp(gsum, efp, Wfij, b_edge, attn_flat, G):
    be = 1024
    grid = (EPAD // be,)
    return pl.pallas_call(
        functools.partial(_edge_body, be),
        grid=grid,
        in_specs=[
            pl.BlockSpec((be, HFE), lambda i: (i, 0)),
            pl.BlockSpec((be, DE), lambda i: (i, 0)),
            pl.BlockSpec((DE, HFE), lambda i: (0, 0)),
            pl.BlockSpec((1, HFE), lambda i: (0, 0)),
            pl.BlockSpec((1, HFE), lambda i: (0, 0)),
            pl.BlockSpec((HFE, H), lambda i: (0, 0)),
        ],
        out_specs=pl.BlockSpec((be, H), lambda i: (i, 0)),
        out_shape=jax.ShapeDtypeStruct((EPAD, H), f32),
    )(gsum, efp, Wfij, b_edge, attn_flat, G)


def _comb_body(p_ref, out_ref):
    s = p_ref[0] + p_ref[1]
    inv = 1.0 / (s + 1e-16)
    out_ref[...] = jnp.concatenate([inv, inv], axis=1)


def _combine_denom(parts):
    return pl.pallas_call(
        _comb_body,
        in_specs=[pl.BlockSpec((2, N, H), lambda: (0, 0, 0))],
        out_specs=pl.BlockSpec((N, 2 * H), lambda: (0, 0)),
        out_shape=jax.ShapeDtypeStruct((N, 2 * H), f32),
    )(parts)


def _fin_body(p_ref, out_ref):
    out_ref[...] = jnp.concatenate([p_ref[0], p_ref[1]], axis=1)


def _combine_parts(parts):
    bnrows = 2000
    return pl.pallas_call(
        _fin_body,
        grid=(N // bnrows,),
        in_specs=[pl.BlockSpec((2, bnrows, HFE), lambda i: (0, i, 0))],
        out_specs=pl.BlockSpec((bnrows, HFN), lambda i: (i, 0)),
        out_shape=jax.ShapeDtypeStruct((N, HFN), f32),
    )(parts)


def _gsum_body(fni, fnj, src2d, dst2d, out_hbm,
               srcb, dstb, a0, a1, b0, b1, o0, o1, sg0, sg1, sw0, sw1):
    wid = _wid()
    rowb = wid * UNITS
    pltpu.sync_copy(src2d.at[pl.ds(rowb, UNITS)], srcb)
    pltpu.sync_copy(dst2d.at[pl.ds(rowb, UNITS)], dstb)
    abuf = (a0, a1)
    bbuf = (b0, b1)
    obuf = (o0, o1)
    sg = (sg0, sg1)
    sw = (sw0, sw1)
    nr = UNITS // 2

    def descs(r, slot):
        u = 2 * r
        d = []
        for j in range(2):
            d.append(pltpu.make_async_copy(
                fni.at[srcb.at[u + j]],
                abuf[slot].at[pl.ds(j * CHUNK, CHUNK)], sg[slot]))
            d.append(pltpu.make_async_copy(
                fnj.at[dstb.at[u + j]],
                bbuf[slot].at[pl.ds(j * CHUNK, CHUNK)], sg[slot]))
        return d

    def wdesc(r, slot):
        eb = wid * EW + r * (2 * CHUNK)
        return pltpu.make_async_copy(
            obuf[slot], out_hbm.at[pl.ds(eb, 2 * CHUNK)], sw[slot])

    def issue(r, slot):
        for dsc in descs(r, slot):
            dsc.start()

    def process(r, slot):
        for dsc in descs(r, slot):
            dsc.wait()

        @pl.when(r >= 2)
        def _():
            wdesc(r - 2, slot).wait()

        av = abuf[slot]
        bv = bbuf[slot]
        ov = obuf[slot]

        def addrow(i, c):
            for k in range(4):
                sl = pl.ds(k * 16, 16)
                ov[i, sl] = av[i, sl] + bv[i, sl]
            return c

        lax.fori_loop(0, 2 * CHUNK, addrow, 0)
        wdesc(r, slot).start()

    issue(0, 0)
    issue(1, 1)

    def step(g, c):
        for s in range(2):
            r = 2 * g + s
            process(r, s)

            @pl.when(r + 2 < nr)
            def _():
                issue(r + 2, s)
        return c

    lax.fori_loop(0, nr // 2, step, 0)
    wdesc(nr - 2, 0).wait()
    wdesc(nr - 1, 1).wait()


def _gather_sum(fni, fnj, src2d, dst2d):
    fn = pl.kernel(
        _gsum_body,
        out_type=jax.ShapeDtypeStruct((EPAD, HFE), f32),
        mesh=_MESH,
        compiler_params=_SC_PARAMS,
        scratch_types=[
            pltpu.VMEM((UNITS, CHUNK), i32),
            pltpu.VMEM((UNITS, CHUNK), i32),
            pltpu.VMEM((2 * CHUNK, HFE), f32),
            pltpu.VMEM((2 * CHUNK, HFE), f32),
            pltpu.VMEM((2 * CHUNK, HFE), f32),
            pltpu.VMEM((2 * CHUNK, HFE), f32),
            pltpu.VMEM((2 * CHUNK, HFE), f32),
            pltpu.VMEM((2 * CHUNK, HFE), f32),
            pltpu.SemaphoreType.DMA,
            pltpu.SemaphoreType.DMA,
            pltpu.SemaphoreType.DMA,
            pltpu.SemaphoreType.DMA,
        ],
    )
    return fn(fni, fnj, src2d, dst2d)


def _scat8_body(eexp_hbm, dst2d, zeros_hbm, out_hbm, dstb, eb, acc,
                s0, s1, s2, s3, s4, s5, s6, s7):
    sems = (s0, s1, s2, s3, s4, s5, s6, s7)
    c = lax.axis_index("c")
    s = lax.axis_index("s")
    wid = s * NC + c
    rb = s * ROWS_PER_SUB
    pltpu.sync_copy(zeros_hbm.at[pl.ds(rb, ROWS_PER_SUB)],
                    acc.at[pl.ds(rb, ROWS_PER_SUB)])
    pltpu.sync_copy(dst2d.at[pl.ds(wid * UNITS, UNITS)], dstb)
    pltpu.sync_copy(eexp_hbm.at[pl.ds(wid * EW, EW)], eb)
    plsc.subcore_barrier()

    def sdesc(u, sem):
        return pltpu.make_async_copy(
            eb.at[pl.ds(u * CHUNK, CHUNK)], acc.at[dstb.at[u]], sem)

    def rnd(r, carry):
        for j in range(8):
            sdesc(r * 8 + j, sems[j]).start(add=True)
        for j in range(8):
            sdesc(r * 8 + j, sems[j]).wait()
        return carry

    lax.fori_loop(0, UNITS // 8, rnd, 0)
    plsc.subcore_barrier()
    pltpu.sync_copy(acc.at[pl.ds(rb, ROWS_PER_SUB)],
                    out_hbm.at[c, pl.ds(rb, ROWS_PER_SUB)])


def _denom_partials(eexp, dst2d, zeros8):
    fn = pl.kernel(
        _scat8_body,
        out_type=jax.ShapeDtypeStruct((NC, N, H), f32),
        mesh=_MESH,
        compiler_params=_SC_PARAMS,
        scratch_types=[
            pltpu.VMEM((UNITS, CHUNK), i32),
            pltpu.VMEM((EW, H), f32),
            pltpu.VMEM_SHARED((N, H), f32),
            pltpu.SemaphoreType.DMA,
            pltpu.SemaphoreType.DMA,
            pltpu.SemaphoreType.DMA,
            pltpu.SemaphoreType.DMA,
            pltpu.SemaphoreType.DMA,
            pltpu.SemaphoreType.DMA,
            pltpu.SemaphoreType.DMA,
            pltpu.SemaphoreType.DMA,
        ],
    )
    return fn(eexp, dst2d, zeros8)


def _msg_body(hcat_hbm, ivd_hbm, eexp2_hbm, src2d, dst2d, zeros_hbm, out_hbm,
              srcb, dstb, hb0, hb1, ivb0, ivb1, ebf0, ebf1, mb0, mb1, acc,
              sh0, sh1, si0, si1, se0, se1, sw0, sw1):
    c = lax.axis_index("c")
    s = lax.axis_index("s")
    rb = s * ROWS_PER_SUB
    pltpu.sync_copy(zeros_hbm.at[pl.ds(rb, ROWS_PER_SUB)],
                    acc.at[pl.ds(rb, ROWS_PER_SUB)])
    rowb = s * MUNITS
    pltpu.sync_copy(src2d.at[pl.ds(rowb, MUNITS)], srcb)
    pltpu.sync_copy(dst2d.at[pl.ds(rowb, MUNITS)], dstb)
    coff = c * N

    def offrow(i, cc):
        for k in range(CHUNK // 16):
            sl = pl.ds(k * 16, 16)
            srcb[i, sl] = srcb[i, sl] + coff
        return cc

    lax.fori_loop(0, MUNITS, offrow, 0)
    plsc.subcore_barrier()

    hb = (hb0, hb1)
    ivb = (ivb0, ivb1)
    ebf = (ebf0, ebf1)
    mb = (mb0, mb1)
    sh = (sh0, sh1)
    si = (si0, si1)
    se = (se0, se1)
    sw = (sw0, sw1)

    lane = lax.broadcasted_iota(i32, (16,), 0)
    lo_half = lane < 8
    hbase = c * 4
    dnums = lax.GatherDimensionNumbers(
        offset_dims=(), collapsed_slice_dims=(0,), start_index_map=(0,))

    def _splat(v, k):
        idx = jnp.broadcast_to(jnp.reshape(k, (1, 1)), (16, 1))
        return lax.gather(v, idx, dnums, slice_sizes=(1,),
                          mode=lax.GatherScatterMode.PROMISE_IN_BOUNDS)

    def descs(u, slot):
        return [
            pltpu.make_async_copy(hcat_hbm.at[srcb.at[u]], hb[slot], sh[slot]),
            pltpu.make_async_copy(ivd_hbm.at[dstb.at[u]], ivb[slot], si[slot]),
            pltpu.make_async_copy(
                eexp2_hbm.at[pl.ds(rowb * (CHUNK // 2) + u * (CHUNK // 2),
                                   CHUNK // 2)],
                ebf[slot], se[slot]),
        ]

    def wdesc(u, slot):
        return pltpu.make_async_copy(mb[slot], acc.at[dstb.at[u]], sw[slot])

    def issue(u, slot):
        for dsc in descs(u, slot):
            dsc.start()

    def process(u, slot):
        for dsc in descs(u, slot):
            dsc.wait()
        hbv = hb[slot]
        ivv = ivb[slot]
        ebv = ebf[slot]
        mbv = mb[slot]

        @pl.when(u >= 2)
        def _():
            wdesc(u - 2, slot).wait()

        def pair(p, cc):
            ep = ebv[p, :]
            iv0 = ivv[2 * p, :]
            iv1 = ivv[2 * p + 1, :]
            ap = jnp.where(lo_half, iv0, iv1) * ep
            for k in range(4):
                b0 = _splat(ap, hbase + k)
                sl = pl.ds(k * 16, 16)
                mbv[2 * p, sl] = hbv[2 * p, sl] * b0
                b1 = _splat(ap, hbase + (8 + k))
                mbv[2 * p + 1, sl] = hbv[2 * p + 1, sl] * b1
            return cc

        lax.fori_loop(0, CHUNK // 2, pair, 0)
        wdesc(u, slot).start(add=True)

    issue(0, 0)
    issue(1, 1)

    def step(g, carry):
        for s2 in range(2):
            u = 2 * g + s2
            process(u, s2)

            @pl.when(u + 2 < MUNITS)
            def _():
                issue(u + 2, s2)
        return carry

    lax.fori_loop(0, MUNITS // 2, step, 0)
    wdesc(MUNITS - 2, 0).wait()
    wdesc(MUNITS - 1, 1).wait()
    plsc.subcore_barrier()
    pltpu.sync_copy(acc.at[pl.ds(rb, ROWS_PER_SUB)],
                    out_hbm.at[c, pl.ds(rb, ROWS_PER_SUB)])


def _message_partials(hcat, ivd, eexp2, src2d, dst2d, zerosN):
    fn = pl.kernel(
        _msg_body,
        out_type=jax.ShapeDtypeStruct((NC, N, HFE), f32),
        mesh=_MESH,
        compiler_params=_SC_PARAMS,
        scratch_types=[
            pltpu.VMEM((MUNITS, CHUNK), i32),
            pltpu.VMEM((MUNITS, CHUNK), i32),
            pltpu.VMEM((CHUNK, HFE), f32),
            pltpu.VMEM((CHUNK, HFE), f32),
            pltpu.VMEM((CHUNK, 2 * H), f32),
            pltpu.VMEM((CHUNK, 2 * H), f32),
            pltpu.VMEM((CHUNK // 2, 16), f32),
            pltpu.VMEM((CHUNK // 2, 16), f32),
            pltpu.VMEM((CHUNK, HFE), f32),
            pltpu.VMEM((CHUNK, HFE), f32),
            pltpu.VMEM_SHARED((N, HFE), f32),
            pltpu.SemaphoreType.DMA,
            pltpu.SemaphoreType.DMA,
            pltpu.SemaphoreType.DMA,
            pltpu.SemaphoreType.DMA,
            pltpu.SemaphoreType.DMA,
            pltpu.SemaphoreType.DMA,
            pltpu.SemaphoreType.DMA,
            pltpu.SemaphoreType.DMA,
        ],
    )
    return fn(hcat, ivd, eexp2, src2d, dst2d, zerosN)


def _layer(hin, efp, src2d, dst2d, zeros8, zerosN, G,
           Wn, bn, Wni, Wnj, Wfij, attn, be_bias):
    fni, fnj, h_lo, h_hi = _node_matmuls(hin, Wni, Wnj, Wn, bn.reshape(1, HFN))
    hcat = jnp.concatenate([h_lo, h_hi], axis=0)
    gsum = _gather_sum(fni, fnj, src2d, dst2d)
    eexp = _edge_eexp(gsum, efp, Wfij, be_bias.reshape(1, HFE),
                      attn.reshape(1, HFE), G)
    dparts = _denom_partials(eexp, dst2d, zeros8)
    ivd = _combine_denom(dparts)
    eexp2 = eexp.reshape(EPAD // 2, 2 * H)
    return _message_partials(hcat, ivd, eexp2, src2d, dst2d, zerosN)


def kernel(n_feat, e_feat, edge_index, W_node0, b_node0, W_ni0, W_nj0,
           W_fij0, attn0, b_edge0, W_node1, b_node1, W_ni1, W_nj1, W_fij1,
           attn1, b_edge1):
    src = edge_index[0].astype(i32)
    dst = edge_index[1].astype(i32)
    pad = EPAD - E
    src2d = jnp.concatenate([src, jnp.zeros((pad,), i32)]).reshape(NROW, CHUNK)
    dst2d = jnp.concatenate([dst, jnp.zeros((pad,), i32)]).reshape(NROW, CHUNK)
    efp = jnp.concatenate([e_feat, jnp.zeros((pad, DE), f32)])
    zeros8 = jnp.zeros((N, H), f32)
    zerosN = jnp.zeros((N, HFE), f32)
    G = jnp.asarray(np.kron(np.eye(H), np.ones((FE, 1))), f32)

    parts0 = _layer(n_feat[None], efp, src2d, dst2d, zeros8, zerosN, G,
                    W_node0, b_node0, W_ni0, W_nj0, W_fij0, attn0, b_edge0)
    parts1 = _layer(parts0, efp, src2d, dst2d, zeros8, zerosN, G,
                    W_node1, b_node1, W_ni1, W_nj1, W_fij1, attn1, b_edge1)
    return _combine_parts(parts1)


# speedup vs baseline: 30.7047x; 1.0087x over previous
import functools

import jax
import jax.numpy as jnp
import numpy as np
from jax import lax
from jax.experimental import pallas as pl
from jax.experimental.pallas import tpu as pltpu
from jax.experimental.pallas import tpu_sc as plsc

N = 10000
E = 320000
D = 128
DE = 16
H = 8
FE = 8
FN = 16
HFE = H * FE
HFN = H * FN

NC = 2
NS = 16
NW = NC * NS
CHUNK = 128
UNITS = 80
EW = CHUNK * UNITS
EPAD = EW * NW
NROW = EPAD // CHUNK
ROWS_PER_SUB = N // NS
MUNITS = NROW // NS

_MESH = plsc.VectorSubcoreMesh(
    core_axis_name="c", subcore_axis_name="s", num_cores=NC, num_subcores=NS)
_SC_PARAMS = pltpu.CompilerParams(use_tc_tiling_on_sc=False)

f32 = jnp.float32
i32 = jnp.int32


def _wid():
    return lax.axis_index("s") * NC + lax.axis_index("c")


def _mm_body(colsplit, hin_ref, dp_ref, gx_ref, wni_ref, wnj_ref, wn_ref,
             bn_ref, fni_ref, fnj_ref, hlo_ref, hhi_ref):
    if colsplit:
        dinv = 1.0 / (dp_ref[0] + dp_ref[1] + 1e-16)
        expand = jnp.dot(dinv, gx_ref[...], preferred_element_type=f32)
        x = jnp.concatenate([hin_ref[0], hin_ref[1]], axis=1) * expand
    else:
        x = hin_ref[0]
    fni_ref[...] = jnp.dot(x, wni_ref[...], preferred_element_type=f32)
    fnj_ref[...] = jnp.dot(x, wnj_ref[...], preferred_element_type=f32)
    hfull = jnp.dot(x, wn_ref[...], preferred_element_type=f32) + bn_ref[...]
    hlo_ref[...] = hfull[:, :HFE]
    hhi_ref[...] = hfull[:, HFE:]


def _node_matmuls(hin, dparts, Gx, Wni, Wnj, Wn, bn):
    colsplit = hin.shape[0] == 2
    din = hin.shape[2]
    bnrows = 400
    grid = (N // bnrows,)
    return pl.pallas_call(
        functools.partial(_mm_body, colsplit),
        grid=grid,
        in_specs=[
            pl.BlockSpec((hin.shape[0], bnrows, din), lambda i: (0, i, 0)),
            pl.BlockSpec((2, bnrows, H), lambda i: (0, i, 0)),
            pl.BlockSpec((H, HFN), lambda i: (0, 0)),
            pl.BlockSpec((D, HFE), lambda i: (0, 0)),
            pl.BlockSpec((D, HFE), lambda i: (0, 0)),
            pl.BlockSpec((D, HFN), lambda i: (0, 0)),
            pl.BlockSpec((1, HFN), lambda i: (0, 0)),
        ],
        out_specs=[
            pl.BlockSpec((bnrows, HFE), lambda i: (i, 0)),
            pl.BlockSpec((bnrows, HFE), lambda i: (i, 0)),
            pl.BlockSpec((bnrows, HFE), lambda i: (i, 0)),
            pl.BlockSpec((bnrows, HFE), lambda i: (i, 0)),
        ],
        out_shape=[
            jax.ShapeDtypeStruct((N, HFE), f32),
            jax.ShapeDtypeStruct((N, HFE), f32),
            jax.ShapeDtypeStruct((N, HFE), f32),
            jax.ShapeDtypeStruct((N, HFE), f32),
        ],
    )(hin, dparts, Gx, Wni, Wnj, Wn, bn)


def _edge_body(be, gs_ref, ef_ref, wf_ref, bias_ref, attn_ref, g_ref,
               out_ref):
    z = gs_ref[...]
    z = z + jnp.dot(ef_ref[...], wf_ref[...], preferred_element_type=f32)
    z = z + bias_ref[...]
    z = jnp.where(z >= 0, z, 0.01 * z)
    e = jnp.dot(z * attn_ref[...], g_ref[...], preferred_element_type=f32)
    pid = pl.program_id(0)
    eid = pid * be + lax.broadcasted_iota(i32, (be, H), 0)
    out_ref[...] = jnp.where(eid < E, jnp.exp(e), 0.0)


def _edge_eexp(gsum, efp, Wfij, b_edge, attn_flat, G):
    be = 1024
    grid = (EPAD // be,)
    return pl.pallas_call(
        functools.partial(_edge_body, be),
        grid=grid,
        in_specs=[
            pl.BlockSpec((be, HFE), lambda i: (i, 0)),
            pl.BlockSpec((be, DE), lambda i: (i, 0)),
            pl.BlockSpec((DE, HFE), lambda i: (0, 0)),
            pl.BlockSpec((1, HFE), lambda i: (0, 0)),
            pl.BlockSpec((1, HFE), lambda i: (0, 0)),
            pl.BlockSpec((HFE, H), lambda i: (0, 0)),
        ],
        out_specs=pl.BlockSpec((be, H), lambda i: (i, 0)),
        out_shape=jax.ShapeDtypeStruct((EPAD, H), f32),
    )(gsum, efp, Wfij, b_edge, attn_flat, G)


def _fin_body(p_ref, dp_ref, gx_ref, out_ref):
    dinv = 1.0 / (dp_ref[0] + dp_ref[1] + 1e-16)
    expand = jnp.dot(dinv, gx_ref[...], preferred_element_type=f32)
    out_ref[...] = jnp.concatenate([p_ref[0], p_ref[1]], axis=1) * expand


def _combine_parts(parts, dparts, Gx):
    bnrows = 2000
    return pl.pallas_call(
        _fin_body,
        grid=(N // bnrows,),
        in_specs=[
            pl.BlockSpec((2, bnrows, HFE), lambda i: (0, i, 0)),
            pl.BlockSpec((2, bnrows, H), lambda i: (0, i, 0)),
            pl.BlockSpec((H, HFN), lambda i: (0, 0)),
        ],
        out_specs=pl.BlockSpec((bnrows, HFN), lambda i: (i, 0)),
        out_shape=jax.ShapeDtypeStruct((N, HFN), f32),
    )(parts, dparts, Gx)


def _gsum_body(fni, fnj, src2d, dst2d, out_hbm,
               srcb, dstb, a0, a1, b0, b1, o0, o1, sg0, sg1, sw0, sw1):
    wid = _wid()
    rowb = wid * UNITS
    pltpu.sync_copy(src2d.at[pl.ds(rowb, UNITS)], srcb)
    pltpu.sync_copy(dst2d.at[pl.ds(rowb, UNITS)], dstb)
    abuf = (a0, a1)
    bbuf = (b0, b1)
    obuf = (o0, o1)
    sg = (sg0, sg1)
    sw = (sw0, sw1)
    nr = UNITS // 2

    def descs(r, slot):
        u = 2 * r
        d = []
        for j in range(2):
            d.append(pltpu.make_async_copy(
                fni.at[srcb.at[u + j]],
                abuf[slot].at[pl.ds(j * CHUNK, CHUNK)], sg[slot]))
            d.append(pltpu.make_async_copy(
                fnj.at[dstb.at[u + j]],
                bbuf[slot].at[pl.ds(j * CHUNK, CHUNK)], sg[slot]))
        return d

    def wdesc(r, slot):
        eb = wid * EW + r * (2 * CHUNK)
        return pltpu.make_async_copy(
            obuf[slot], out_hbm.at[pl.ds(eb, 2 * CHUNK)], sw[slot])

    def issue(r, slot):
        for dsc in descs(r, slot):
            dsc.start()

    def process(r, slot):
        for dsc in descs(r, slot):
            dsc.wait()

        @pl.when(r >= 2)
        def _():
            wdesc(r - 2, slot).wait()

        av = abuf[slot]
        bv = bbuf[slot]
        ov = obuf[slot]

        def addrow(i, c):
            for k in range(4):
                sl = pl.ds(k * 16, 16)
                ov[i, sl] = av[i, sl] + bv[i, sl]
            return c

        lax.fori_loop(0, 2 * CHUNK, addrow, 0)
        wdesc(r, slot).start()

    issue(0, 0)
    issue(1, 1)

    def step(g, c):
        for s in range(2):
            r = 2 * g + s
            process(r, s)

            @pl.when(r + 2 < nr)
            def _():
                issue(r + 2, s)
        return c

    lax.fori_loop(0, nr // 2, step, 0)
    wdesc(nr - 2, 0).wait()
    wdesc(nr - 1, 1).wait()


def _gather_sum(fni, fnj, src2d, dst2d):
    fn = pl.kernel(
        _gsum_body,
        out_type=jax.ShapeDtypeStruct((EPAD, HFE), f32),
        mesh=_MESH,
        compiler_params=_SC_PARAMS,
        scratch_types=[
            pltpu.VMEM((UNITS, CHUNK), i32),
            pltpu.VMEM((UNITS, CHUNK), i32),
            pltpu.VMEM((2 * CHUNK, HFE), f32),
            pltpu.VMEM((2 * CHUNK, HFE), f32),
            pltpu.VMEM((2 * CHUNK, HFE), f32),
            pltpu.VMEM((2 * CHUNK, HFE), f32),
            pltpu.VMEM((2 * CHUNK, HFE), f32),
            pltpu.VMEM((2 * CHUNK, HFE), f32),
            pltpu.SemaphoreType.DMA,
            pltpu.SemaphoreType.DMA,
            pltpu.SemaphoreType.DMA,
            pltpu.SemaphoreType.DMA,
        ],
    )
    return fn(fni, fnj, src2d, dst2d)


def _scat8_body(eexp_hbm, dst2d, zeros_hbm, out_hbm, dstb, eb, acc,
                s0, s1, s2, s3, s4, s5, s6, s7):
    sems = (s0, s1, s2, s3, s4, s5, s6, s7)
    c = lax.axis_index("c")
    s = lax.axis_index("s")
    wid = s * NC + c
    rb = s * ROWS_PER_SUB
    pltpu.sync_copy(zeros_hbm.at[pl.ds(rb, ROWS_PER_SUB)],
                    acc.at[pl.ds(rb, ROWS_PER_SUB)])
    pltpu.sync_copy(dst2d.at[pl.ds(wid * UNITS, UNITS)], dstb)
    pltpu.sync_copy(eexp_hbm.at[pl.ds(wid * EW, EW)], eb)
    plsc.subcore_barrier()

    def sdesc(u, sem):
        return pltpu.make_async_copy(
            eb.at[pl.ds(u * CHUNK, CHUNK)], acc.at[dstb.at[u]], sem)

    def rnd(r, carry):
        for j in range(8):
            sdesc(r * 8 + j, sems[j]).start(add=True)
        for j in range(8):
            sdesc(r * 8 + j, sems[j]).wait()
        return carry

    lax.fori_loop(0, UNITS // 8, rnd, 0)
    plsc.subcore_barrier()
    pltpu.sync_copy(acc.at[pl.ds(rb, ROWS_PER_SUB)],
                    out_hbm.at[c, pl.ds(rb, ROWS_PER_SUB)])


def _denom_partials(eexp, dst2d, zeros8):
    fn = pl.kernel(
        _scat8_body,
        out_type=jax.ShapeDtypeStruct((NC, N, H), f32),
        mesh=_MESH,
        compiler_params=_SC_PARAMS,
        scratch_types=[
            pltpu.VMEM((UNITS, CHUNK), i32),
            pltpu.VMEM((EW, H), f32),
            pltpu.VMEM_SHARED((N, H), f32),
            pltpu.SemaphoreType.DMA,
            pltpu.SemaphoreType.DMA,
            pltpu.SemaphoreType.DMA,
            pltpu.SemaphoreType.DMA,
            pltpu.SemaphoreType.DMA,
            pltpu.SemaphoreType.DMA,
            pltpu.SemaphoreType.DMA,
            pltpu.SemaphoreType.DMA,
        ],
    )
    return fn(eexp, dst2d, zeros8)


def _msg_body(hcat_hbm, eexp2_hbm, src2d, dst2d, zeros_hbm, out_hbm,
              srcb, dstb, hb0, hb1, ebf0, ebf1, mb0, mb1, acc,
              sh0, sh1, se0, se1, sw0, sw1):
    c = lax.axis_index("c")
    s = lax.axis_index("s")
    rb = s * ROWS_PER_SUB
    pltpu.sync_copy(zeros_hbm.at[pl.ds(rb, ROWS_PER_SUB)],
                    acc.at[pl.ds(rb, ROWS_PER_SUB)])
    rowb = s * MUNITS
    pltpu.sync_copy(src2d.at[pl.ds(rowb, MUNITS)], srcb)
    pltpu.sync_copy(dst2d.at[pl.ds(rowb, MUNITS)], dstb)
    coff = c * N

    def offrow(i, cc):
        for k in range(CHUNK // 16):
            sl = pl.ds(k * 16, 16)
            srcb[i, sl] = srcb[i, sl] + coff
        return cc

    lax.fori_loop(0, MUNITS, offrow, 0)
    plsc.subcore_barrier()

    hb = (hb0, hb1)
    ebf = (ebf0, ebf1)
    mb = (mb0, mb1)
    sh = (sh0, sh1)
    se = (se0, se1)
    sw = (sw0, sw1)

    hbase = c * 4
    dnums = lax.GatherDimensionNumbers(
        offset_dims=(), collapsed_slice_dims=(0,), start_index_map=(0,))

    def _splat(v, k):
        idx = jnp.broadcast_to(jnp.reshape(k, (1, 1)), (16, 1))
        return lax.gather(v, idx, dnums, slice_sizes=(1,),
                          mode=lax.GatherScatterMode.PROMISE_IN_BOUNDS)

    def descs(u, slot):
        return [
            pltpu.make_async_copy(hcat_hbm.at[srcb.at[u]], hb[slot], sh[slot]),
            pltpu.make_async_copy(
                eexp2_hbm.at[pl.ds(rowb * (CHUNK // 2) + u * (CHUNK // 2),
                                   CHUNK // 2)],
                ebf[slot], se[slot]),
        ]

    def wdesc(u, slot):
        return pltpu.make_async_copy(mb[slot], acc.at[dstb.at[u]], sw[slot])

    def issue(u, slot):
        for dsc in descs(u, slot):
            dsc.start()

    def process(u, slot):
        for dsc in descs(u, slot):
            dsc.wait()
        hbv = hb[slot]
        ebv = ebf[slot]
        mbv = mb[slot]

        @pl.when(u >= 2)
        def _():
            wdesc(u - 2, slot).wait()

        def pair(p, cc):
            ap = ebv[p, :]
            for k in range(4):
                b0 = _splat(ap, hbase + k)
                sl = pl.ds(k * 16, 16)
                mbv[2 * p, sl] = hbv[2 * p, sl] * b0
                b1 = _splat(ap, hbase + (8 + k))
                mbv[2 * p + 1, sl] = hbv[2 * p + 1, sl] * b1
            return cc

        lax.fori_loop(0, CHUNK // 2, pair, 0)
        wdesc(u, slot).start(add=True)

    issue(0, 0)
    issue(1, 1)

    def step(g, carry):
        for s2 in range(2):
            u = 2 * g + s2
            process(u, s2)

            @pl.when(u + 2 < MUNITS)
            def _():
                issue(u + 2, s2)
        return carry

    lax.fori_loop(0, MUNITS // 2, step, 0)
    wdesc(MUNITS - 2, 0).wait()
    wdesc(MUNITS - 1, 1).wait()
    plsc.subcore_barrier()
    pltpu.sync_copy(acc.at[pl.ds(rb, ROWS_PER_SUB)],
                    out_hbm.at[c, pl.ds(rb, ROWS_PER_SUB)])


def _message_partials(hcat, eexp2, src2d, dst2d, zerosN):
    fn = pl.kernel(
        _msg_body,
        out_type=jax.ShapeDtypeStruct((NC, N, HFE), f32),
        mesh=_MESH,
        compiler_params=_SC_PARAMS,
        scratch_types=[
            pltpu.VMEM((MUNITS, CHUNK), i32),
            pltpu.VMEM((MUNITS, CHUNK), i32),
            pltpu.VMEM((CHUNK, HFE), f32),
            pltpu.VMEM((CHUNK, HFE), f32),
            pltpu.VMEM((CHUNK // 2, 16), f32),
            pltpu.VMEM((CHUNK // 2, 16), f32),
            pltpu.VMEM((CHUNK, HFE), f32),
            pltpu.VMEM((CHUNK, HFE), f32),
            pltpu.VMEM_SHARED((N, HFE), f32),
            pltpu.SemaphoreType.DMA,
            pltpu.SemaphoreType.DMA,
            pltpu.SemaphoreType.DMA,
            pltpu.SemaphoreType.DMA,
            pltpu.SemaphoreType.DMA,
            pltpu.SemaphoreType.DMA,
        ],
    )
    return fn(hcat, eexp2, src2d, dst2d, zerosN)


def _layer(hin, dparts_prev, efp, src2d, dst2d, zeros8, zerosN, G, Gx,
           Wn, bn, Wni, Wnj, Wfij, attn, be_bias):
    fni, fnj, h_lo, h_hi = _node_matmuls(hin, dparts_prev, Gx, Wni, Wnj, Wn,
                                         bn.reshape(1, HFN))
    hcat = jnp.concatenate([h_lo, h_hi], axis=0)
    gsum = _gather_sum(fni, fnj, src2d, dst2d)
    eexp = _edge_eexp(gsum, efp, Wfij, be_bias.reshape(1, HFE),
                      attn.reshape(1, HFE), G)
    dparts = _denom_partials(eexp, dst2d, zeros8)
    eexp2 = eexp.reshape(EPAD // 2, 2 * H)
    parts = _message_partials(hcat, eexp2, src2d, dst2d, zerosN)
    return parts, dparts


def kernel(n_feat, e_feat, edge_index, W_node0, b_node0, W_ni0, W_nj0,
           W_fij0, attn0, b_edge0, W_node1, b_node1, W_ni1, W_nj1, W_fij1,
           attn1, b_edge1):
    src = edge_index[0].astype(i32)
    dst = edge_index[1].astype(i32)
    pad = EPAD - E
    src2d = jnp.concatenate([src, jnp.zeros((pad,), i32)]).reshape(NROW, CHUNK)
    dst2d = jnp.concatenate([dst, jnp.zeros((pad,), i32)]).reshape(NROW, CHUNK)
    efp = jnp.concatenate([e_feat, jnp.zeros((pad, DE), f32)])
    zeros8 = jnp.zeros((N, H), f32)
    zerosN = jnp.zeros((N, HFE), f32)
    dzero = jnp.full((2, N, H), 0.5, f32)
    G = jnp.asarray(np.kron(np.eye(H), np.ones((FE, 1))), f32)
    Gx = jnp.asarray(np.kron(np.eye(H), np.ones((1, FN))), f32)

    parts0, dparts0 = _layer(n_feat[None], dzero, efp, src2d, dst2d, zeros8,
                             zerosN, G, Gx, W_node0, b_node0, W_ni0, W_nj0,
                             W_fij0, attn0, b_edge0)
    parts1, dparts1 = _layer(parts0, dparts0, efp, src2d, dst2d, zeros8,
                             zerosN, G, Gx, W_node1, b_node1, W_ni1, W_nj1,
                             W_fij1, attn1, b_edge1)
    return _combine_parts(parts1, dparts1, Gx)


# speedup vs baseline: 31.8948x; 1.0388x over previous
import functools

import jax
import jax.numpy as jnp
import numpy as np
from jax import lax
from jax.experimental import pallas as pl
from jax.experimental.pallas import tpu as pltpu
from jax.experimental.pallas import tpu_sc as plsc

N = 10000
E = 320000
D = 128
DE = 16
H = 8
FE = 8
FN = 16
HFE = H * FE
HFN = H * FN

NC = 2
NS = 16
NW = NC * NS
CHUNK = 128
UNITS = 80
EW = CHUNK * UNITS
EPAD = EW * NW
NROW = EPAD // CHUNK
ROWS_PER_SUB = N // NS
MUNITS = NROW // NS

_MESH = plsc.VectorSubcoreMesh(
    core_axis_name="c", subcore_axis_name="s", num_cores=NC, num_subcores=NS)
_SC_PARAMS = pltpu.CompilerParams(use_tc_tiling_on_sc=False)

f32 = jnp.float32
i32 = jnp.int32


def _wid():
    return lax.axis_index("s") * NC + lax.axis_index("c")


def _mm_body(colsplit, hin_ref, gx_ref, wni_ref, wnj_ref, wn_ref,
             bn_ref, fni_ref, fnj_ref, hlo_ref, hhi_ref):
    if colsplit:
        dcat = jnp.concatenate([hin_ref[0, :, 64:68], hin_ref[1, :, 64:68]],
                               axis=1)
        dinv = 1.0 / (dcat + 1e-16)
        expand = jnp.dot(dinv, gx_ref[...], preferred_element_type=f32)
        x = jnp.concatenate([hin_ref[0, :, :HFE], hin_ref[1, :, :HFE]],
                            axis=1) * expand
    else:
        x = hin_ref[0]
    fni_ref[...] = jnp.dot(x, wni_ref[...], preferred_element_type=f32)
    fnj_ref[...] = jnp.dot(x, wnj_ref[...], preferred_element_type=f32)
    hfull = jnp.dot(x, wn_ref[...], preferred_element_type=f32) + bn_ref[...]
    hlo_ref[...] = hfull[:, :HFE]
    hhi_ref[...] = hfull[:, HFE:]


def _node_matmuls(hin, Gx, Wni, Wnj, Wn, bn):
    colsplit = hin.shape[0] == 2
    din = hin.shape[2]
    bnrows = 400
    grid = (N // bnrows,)
    return pl.pallas_call(
        functools.partial(_mm_body, colsplit),
        grid=grid,
        in_specs=[
            pl.BlockSpec((hin.shape[0], bnrows, din), lambda i: (0, i, 0)),
            pl.BlockSpec((H, HFN), lambda i: (0, 0)),
            pl.BlockSpec((D, HFE), lambda i: (0, 0)),
            pl.BlockSpec((D, HFE), lambda i: (0, 0)),
            pl.BlockSpec((D, HFN), lambda i: (0, 0)),
            pl.BlockSpec((1, HFN), lambda i: (0, 0)),
        ],
        out_specs=[
            pl.BlockSpec((bnrows, HFE), lambda i: (i, 0)),
            pl.BlockSpec((bnrows, HFE), lambda i: (i, 0)),
            pl.BlockSpec((bnrows, HFE), lambda i: (i, 0)),
            pl.BlockSpec((bnrows, HFE), lambda i: (i, 0)),
        ],
        out_shape=[
            jax.ShapeDtypeStruct((N, HFE), f32),
            jax.ShapeDtypeStruct((N, HFE), f32),
            jax.ShapeDtypeStruct((N, HFE), f32),
            jax.ShapeDtypeStruct((N, HFE), f32),
        ],
    )(hin, Gx, Wni, Wnj, Wn, bn)


def _edge_body(be, gs_ref, ef_ref, wf_ref, bias_ref, attn_ref, g_ref,
               out_ref):
    z = gs_ref[...]
    z = z + jnp.dot(ef_ref[...], wf_ref[...], preferred_element_type=f32)
    z = z + bias_ref[...]
    z = jnp.where(z >= 0, z, 0.01 * z)
    e = jnp.dot(z * attn_ref[...], g_ref[...], preferred_element_type=f32)
    pid = pl.program_id(0)
    eid = pid * be + lax.broadcasted_iota(i32, (be, H), 0)
    out_ref[...] = jnp.where(eid < E, jnp.exp(e), 0.0).T


def _edge_eexp(gsum, efp, Wfij, b_edge, attn_flat, G):
    be = 1024
    grid = (EPAD // be,)
    elast = E // be
    return pl.pallas_call(
        functools.partial(_edge_body, be),
        grid=grid,
        in_specs=[
            pl.BlockSpec((be, HFE), lambda i: (i, 0)),
            pl.BlockSpec((be, DE), lambda i: (jnp.minimum(i, elast), 0)),
            pl.BlockSpec((DE, HFE), lambda i: (0, 0)),
            pl.BlockSpec((1, HFE), lambda i: (0, 0)),
            pl.BlockSpec((1, HFE), lambda i: (0, 0)),
            pl.BlockSpec((HFE, H), lambda i: (0, 0)),
        ],
        out_specs=pl.BlockSpec((H, be), lambda i: (0, i)),
        out_shape=jax.ShapeDtypeStruct((H, EPAD), f32),
    )(gsum, efp, Wfij, b_edge, attn_flat, G)


def _fin_body(p_ref, gx_ref, out_ref):
    dcat = jnp.concatenate([p_ref[0, :, 64:68], p_ref[1, :, 64:68]], axis=1)
    dinv = 1.0 / (dcat + 1e-16)
    expand = jnp.dot(dinv, gx_ref[...], preferred_element_type=f32)
    out_ref[...] = jnp.concatenate([p_ref[0, :, :HFE], p_ref[1, :, :HFE]],
                                   axis=1) * expand


def _combine_parts(parts, Gx):
    bnrows = 2000
    return pl.pallas_call(
        _fin_body,
        grid=(N // bnrows,),
        in_specs=[
            pl.BlockSpec((2, bnrows, 80), lambda i: (0, i, 0)),
            pl.BlockSpec((H, HFN), lambda i: (0, 0)),
        ],
        out_specs=pl.BlockSpec((bnrows, HFN), lambda i: (i, 0)),
        out_shape=jax.ShapeDtypeStruct((N, HFN), f32),
    )(parts, Gx)


def _gsum_body(fni, fnj, src2d, dst2d, out_hbm,
               srcb, dstb, a0, a1, b0, b1, o0, o1, sg0, sg1, sw0, sw1):
    wid = _wid()
    rowb = wid * UNITS
    pltpu.sync_copy(src2d.at[pl.ds(rowb, UNITS)], srcb)
    pltpu.sync_copy(dst2d.at[pl.ds(rowb, UNITS)], dstb)
    abuf = (a0, a1)
    bbuf = (b0, b1)
    obuf = (o0, o1)
    sg = (sg0, sg1)
    sw = (sw0, sw1)
    nr = UNITS // 2

    def descs(r, slot):
        u = 2 * r
        d = []
        for j in range(2):
            d.append(pltpu.make_async_copy(
                fni.at[srcb.at[u + j]],
                abuf[slot].at[pl.ds(j * CHUNK, CHUNK)], sg[slot]))
            d.append(pltpu.make_async_copy(
                fnj.at[dstb.at[u + j]],
                bbuf[slot].at[pl.ds(j * CHUNK, CHUNK)], sg[slot]))
        return d

    def wdesc(r, slot):
        eb = wid * EW + r * (2 * CHUNK)
        return pltpu.make_async_copy(
            obuf[slot], out_hbm.at[pl.ds(eb, 2 * CHUNK)], sw[slot])

    def issue(r, slot):
        for dsc in descs(r, slot):
            dsc.start()

    def process(r, slot):
        for dsc in descs(r, slot):
            dsc.wait()

        @pl.when(r >= 2)
        def _():
            wdesc(r - 2, slot).wait()

        av = abuf[slot]
        bv = bbuf[slot]
        ov = obuf[slot]

        def addrow(i, c):
            for k in range(4):
                sl = pl.ds(k * 16, 16)
                ov[i, sl] = av[i, sl] + bv[i, sl]
            return c

        lax.fori_loop(0, 2 * CHUNK, addrow, 0)
        wdesc(r, slot).start()

    issue(0, 0)
    issue(1, 1)

    def step(g, c):
        for s in range(2):
            r = 2 * g + s
            process(r, s)

            @pl.when(r + 2 < nr)
            def _():
                issue(r + 2, s)
        return c

    lax.fori_loop(0, nr // 2, step, 0)
    wdesc(nr - 2, 0).wait()
    wdesc(nr - 1, 1).wait()


def _gather_sum(fni, fnj, src2d, dst2d):
    fn = pl.kernel(
        _gsum_body,
        out_type=jax.ShapeDtypeStruct((EPAD, HFE), f32),
        mesh=_MESH,
        compiler_params=_SC_PARAMS,
        scratch_types=[
            pltpu.VMEM((UNITS, CHUNK), i32),
            pltpu.VMEM((UNITS, CHUNK), i32),
            pltpu.VMEM((2 * CHUNK, HFE), f32),
            pltpu.VMEM((2 * CHUNK, HFE), f32),
            pltpu.VMEM((2 * CHUNK, HFE), f32),
            pltpu.VMEM((2 * CHUNK, HFE), f32),
            pltpu.VMEM((2 * CHUNK, HFE), f32),
            pltpu.VMEM((2 * CHUNK, HFE), f32),
            pltpu.SemaphoreType.DMA,
            pltpu.SemaphoreType.DMA,
            pltpu.SemaphoreType.DMA,
            pltpu.SemaphoreType.DMA,
        ],
    )
    return fn(fni, fnj, src2d, dst2d)


def _msg_body(hcat_hbm, eT_hbm, src2d, dst2d, zeros_hbm, out_hbm,
              srcb, dstb, hb0, hb1, ebf0, ebf1, mb0, mb1, acc,
              sh0, sh1, se0, se1, sw0, sw1):
    c = lax.axis_index("c")
    s = lax.axis_index("s")
    rb = s * ROWS_PER_SUB
    pltpu.sync_copy(zeros_hbm.at[pl.ds(rb, ROWS_PER_SUB)],
                    acc.at[pl.ds(rb, ROWS_PER_SUB)])
    rowb = s * MUNITS
    pltpu.sync_copy(src2d.at[pl.ds(rowb, MUNITS)], srcb)
    pltpu.sync_copy(dst2d.at[pl.ds(rowb, MUNITS)], dstb)
    coff = c * N

    def offrow(i, cc):
        for k in range(CHUNK // 16):
            sl = pl.ds(k * 16, 16)
            srcb[i, sl] = srcb[i, sl] + coff
        return cc

    lax.fori_loop(0, MUNITS, offrow, 0)
    plsc.subcore_barrier()

    hb = (hb0, hb1)
    ebf = (ebf0, ebf1)
    mb = (mb0, mb1)
    sh = (sh0, sh1)
    se = (se0, se1)
    sw = (sw0, sw1)

    lane = lax.broadcasted_iota(i32, (16,), 0)
    eqk = [lane == k for k in range(4)]
    hbase = c * 4
    dnums = lax.GatherDimensionNumbers(
        offset_dims=(), collapsed_slice_dims=(0,), start_index_map=(0,))

    def _splat(v, k):
        idx = jnp.broadcast_to(jnp.reshape(k, (1, 1)), (16, 1))
        return lax.gather(v, idx, dnums, slice_sizes=(1,),
                          mode=lax.GatherScatterMode.PROMISE_IN_BOUNDS)

    def descs(u, slot):
        return [
            pltpu.make_async_copy(hcat_hbm.at[srcb.at[u]], hb[slot], sh[slot]),
            pltpu.make_async_copy(
                eT_hbm.at[:, pl.ds((rowb + u) * CHUNK, CHUNK)],
                ebf[slot], se[slot]),
        ]

    def wdesc(u, slot):
        return pltpu.make_async_copy(mb[slot], acc.at[dstb.at[u]], sw[slot])

    def issue(u, slot):
        for dsc in descs(u, slot):
            dsc.start()

    def process(u, slot):
        for dsc in descs(u, slot):
            dsc.wait()

        @pl.when(u >= 2)
        def _():
            wdesc(u - 2, slot).wait()

        hbv = hb[slot]
        ebv = ebf[slot]
        mbv = mb[slot]

        def group(g, cc):
            vk = [ebv[hbase + k, pl.ds(g * 16, 16)] for k in range(4)]

            def edge(e16, c2):
                ed = g * 16 + e16
                d = None
                for k in range(4):
                    b = _splat(vk[k], e16)
                    sl = pl.ds(k * 16, 16)
                    mbv[ed, sl] = hbv[ed, sl] * b
                    d = jnp.where(eqk[k], b, 0.0 if d is None else d)
                mbv[ed, pl.ds(64, 16)] = d
                return c2

            lax.fori_loop(0, 16, edge, 0)
            return cc

        lax.fori_loop(0, CHUNK // 16, group, 0)
        wdesc(u, slot).start(add=True)

    issue(0, 0)
    issue(1, 1)

    def step(g, carry):
        for s2 in range(2):
            u = 2 * g + s2
            process(u, s2)

            @pl.when(u + 2 < MUNITS)
            def _():
                issue(u + 2, s2)
        return carry

    lax.fori_loop(0, MUNITS // 2, step, 0)
    wdesc(MUNITS - 2, 0).wait()
    wdesc(MUNITS - 1, 1).wait()
    plsc.subcore_barrier()
    pltpu.sync_copy(acc.at[pl.ds(rb, ROWS_PER_SUB)],
                    out_hbm.at[c, pl.ds(rb, ROWS_PER_SUB)])


def _message_partials(hcat, eT, src2d, dst2d, zerosN):
    fn = pl.kernel(
        _msg_body,
        out_type=jax.ShapeDtypeStruct((NC, N, 80), f32),
        mesh=_MESH,
        compiler_params=_SC_PARAMS,
        scratch_types=[
            pltpu.VMEM((MUNITS, CHUNK), i32),
            pltpu.VMEM((MUNITS, CHUNK), i32),
            pltpu.VMEM((CHUNK, HFE), f32),
            pltpu.VMEM((CHUNK, HFE), f32),
            pltpu.VMEM((H, CHUNK), f32),
            pltpu.VMEM((H, CHUNK), f32),
            pltpu.VMEM((CHUNK, 80), f32),
            pltpu.VMEM((CHUNK, 80), f32),
            pltpu.VMEM_SHARED((N, 80), f32),
            pltpu.SemaphoreType.DMA,
            pltpu.SemaphoreType.DMA,
            pltpu.SemaphoreType.DMA,
            pltpu.SemaphoreType.DMA,
            pltpu.SemaphoreType.DMA,
            pltpu.SemaphoreType.DMA,
        ],
    )
    return fn(hcat, eT, src2d, dst2d, zerosN)


def _layer(hin, efp, src2d, dst2d, zerosN, G, Gx,
           Wn, bn, Wni, Wnj, Wfij, attn, be_bias):
    fni, fnj, h_lo, h_hi = _node_matmuls(hin, Gx, Wni, Wnj, Wn,
                                         bn.reshape(1, HFN))
    hcat = jnp.concatenate([h_lo, h_hi], axis=0)
    gsum = _gather_sum(fni, fnj, src2d, dst2d)
    eT = _edge_eexp(gsum, efp, Wfij, be_bias.reshape(1, HFE),
                    attn.reshape(1, HFE), G)
    return _message_partials(hcat, eT, src2d, dst2d, zerosN)


def kernel(n_feat, e_feat, edge_index, W_node0, b_node0, W_ni0, W_nj0,
           W_fij0, attn0, b_edge0, W_node1, b_node1, W_ni1, W_nj1, W_fij1,
           attn1, b_edge1):
    src = edge_index[0].astype(i32)
    dst = edge_index[1].astype(i32)
    pad = EPAD - E
    src2d = jnp.concatenate([src, jnp.zeros((pad,), i32)]).reshape(NROW, CHUNK)
    dst2d = jnp.concatenate([dst, jnp.zeros((pad,), i32)]).reshape(NROW, CHUNK)
    zerosN = jnp.zeros((N, 80), f32)
    G = jnp.asarray(np.kron(np.eye(H), np.ones((FE, 1))), f32)
    Gx = jnp.asarray(np.kron(np.eye(H), np.ones((1, FN))), f32)

    parts0 = _layer(n_feat[None], e_feat, src2d, dst2d, zerosN, G, Gx,
                    W_node0, b_node0, W_ni0, W_nj0, W_fij0, attn0, b_edge0)
    parts1 = _layer(parts0, e_feat, src2d, dst2d, zerosN, G, Gx,
                    W_node1, b_node1, W_ni1, W_nj1, W_fij1, attn1, b_edge1)
    return _combine_parts(parts1, Gx)
